# baseline (speedup 1.0000x reference)
import jax
import jax.numpy as jnp
from jax import lax
from jax.experimental import pallas as pl
from jax.experimental.pallas import tpu as pltpu
from jax.experimental.pallas import tpu_sc as plsc

_N_E = 8192
_D = 32
_BETA = 0.25
_B = 8 * 576
_BM = 256
_G = _B // _BM

_NC = 2
_NS = 16
_NW = _NC * _NS
_BPW = _B // _NW


def _main_body(z_ref, embt_ref, zn_ref, idx_ref, ap_ref, ent_ref, hist_ref,
               embn_ref, lesq_ref):
    i = pl.program_id(0)

    @pl.when(i == 0)
    def _init():
        embt = embt_ref[...]
        nsq = jnp.sum(embt * embt, axis=0, keepdims=True)
        embn = embt / jnp.maximum(jnp.sqrt(nsq), 1e-12)
        embn_ref[...] = embn
        lesq_ref[...] = 10.0 * jnp.sum(embn * embn, axis=0, keepdims=True) - 30.0
        ap_ref[...] = jnp.zeros_like(ap_ref)
        ent_ref[...] = jnp.zeros_like(ent_ref)
        hist_ref[...] = jnp.zeros_like(hist_ref)

    z = z_ref[...]
    zn = z / jnp.maximum(jnp.sqrt(jnp.sum(z * z, axis=1, keepdims=True)), 1e-12)
    zn_ref[...] = zn

    dot = jnp.dot(zn, embn_ref[...], preferred_element_type=jnp.float32)
    lc = lesq_ref[...] - 20.0 * dot
    e = jnp.exp(lc)
    lmin = jnp.min(lc, axis=1, keepdims=True)

    ones_n = jnp.ones((_N_E, 1), jnp.float32)
    ones_m = jnp.ones((1, _BM), jnp.float32)
    s = jnp.dot(e, ones_n, preferred_element_type=jnp.float32)
    inv = 1.0 / s
    p = e * inv
    ap_ref[...] += jnp.dot(ones_m, p, preferred_element_type=jnp.float32)
    plc = p * lc
    rowent = jnp.log(s) - jnp.dot(plc, ones_n, preferred_element_type=jnp.float32)
    ent_ref[...] += jnp.sum(rowent).reshape(1, 1)

    cols = lax.broadcasted_iota(jnp.int32, (_BM, _N_E), 1)
    idx = jnp.min(jnp.where(lc == lmin, cols, _N_E), axis=1)
    idx_ref[...] = idx.reshape(1, 1, _BM)
    onehot = jnp.where(cols == idx.reshape(_BM, 1), 1.0, 0.0)
    hist_ref[...] += jnp.dot(ones_m, onehot, preferred_element_type=jnp.float32)


def _main_tc(z_flat, embt):
    return pl.pallas_call(
        _main_body,
        grid=(_G,),
        in_specs=[
            pl.BlockSpec((_BM, _D), lambda i: (i, 0)),
            pl.BlockSpec((_D, _N_E), lambda i: (0, 0)),
        ],
        out_specs=[
            pl.BlockSpec((_BM, _D), lambda i: (i, 0)),
            pl.BlockSpec((1, 1, _BM), lambda i: (i, 0, 0)),
            pl.BlockSpec((1, _N_E), lambda i: (0, 0)),
            pl.BlockSpec((1, 1), lambda i: (0, 0)),
            pl.BlockSpec((1, _N_E), lambda i: (0, 0)),
        ],
        out_shape=[
            jax.ShapeDtypeStruct((_B, _D), jnp.float32),
            jax.ShapeDtypeStruct((_G, 1, _BM), jnp.int32),
            jax.ShapeDtypeStruct((1, _N_E), jnp.float32),
            jax.ShapeDtypeStruct((1, 1), jnp.float32),
            jax.ShapeDtypeStruct((1, _N_E), jnp.float32),
        ],
        scratch_shapes=[
            pltpu.VMEM((_D, _N_E), jnp.float32),
            pltpu.VMEM((1, _N_E), jnp.float32),
        ],
        compiler_params=pltpu.CompilerParams(
            dimension_semantics=("arbitrary",)),
    )(z_flat, embt)


def _sc_gather_body(emb_hbm, idx_hbm, out_hbm, idx_v, rows_v, sem):
    wid = lax.axis_index("s") * _NC + lax.axis_index("c")
    base = wid * _BPW
    pltpu.sync_copy(idx_hbm.at[pl.ds(base, _BPW)], idx_v)
    pltpu.async_copy(emb_hbm.at[idx_v], rows_v, sem).wait()
    pltpu.sync_copy(rows_v, out_hbm.at[pl.ds(base, _BPW)])


def _sc_gather(emb, idx):
    mesh = plsc.VectorSubcoreMesh(core_axis_name="c", subcore_axis_name="s")
    f = pl.kernel(
        _sc_gather_body,
        mesh=mesh,
        out_type=jax.ShapeDtypeStruct((_B, _D), jnp.float32),
        scratch_types=[
            pltpu.VMEM((_BPW,), jnp.int32),
            pltpu.VMEM((_BPW, _D), jnp.float32),
            pltpu.SemaphoreType.DMA,
        ],
        compiler_params=pltpu.CompilerParams(use_tc_tiling_on_sc=False),
    )
    return f(emb, idx)


def _fin_body(zn_ref, zq_ref, ap_ref, ent_ref, hist_ref,
              out_ref, loss_ref, perp_ref, entmin_ref):
    zq = zq_ref[...]
    zqn = zq / jnp.maximum(jnp.sqrt(jnp.sum(zq * zq, axis=1, keepdims=True)),
                           1e-12)
    zn = zn_ref[...]
    out_ref[...] = zn + (zqn - zn)
    diff = zqn - zn
    commit = (1.0 + _BETA) * jnp.mean(diff * diff)
    ap = ap_ref[...] * (1.0 / _B)
    ent_max = -jnp.sum(ap * jnp.log(ap))
    loss_ref[...] = (commit - ent_max).reshape(1, 1)
    probs = hist_ref[...] * (1.0 / _B)
    perp_ref[...] = jnp.exp(-jnp.sum(probs * jnp.log(probs + 1e-10))).reshape(1, 1)
    entmin_ref[...] = ent_ref[...] * (1.0 / _B)


def _finalize(zn, zq, ap_sum, ent_sum, hist):
    return pl.pallas_call(
        _fin_body,
        out_shape=[
            jax.ShapeDtypeStruct((_B, _D), jnp.float32),
            jax.ShapeDtypeStruct((1, 1), jnp.float32),
            jax.ShapeDtypeStruct((1, 1), jnp.float32),
            jax.ShapeDtypeStruct((1, 1), jnp.float32),
        ],
    )(zn, zq, ap_sum, ent_sum, hist)


def kernel(z, emb):
    z_flat = z.reshape(_B, _D)
    embt = emb.T
    zn, idx3, ap_sum, ent_sum, hist = _main_tc(z_flat, embt)
    idx = idx3.reshape(_B)
    zq = _sc_gather(emb, idx)
    zq_out, loss, perp, entmin = _finalize(zn, zq, ap_sum, ent_sum, hist)
    return (zq_out.reshape(z.shape), idx, loss[0, 0], perp[0, 0],
            entmin[0, 0])


# speedup vs baseline: 1.1674x; 1.1674x over previous
import jax
import jax.numpy as jnp
from jax import lax
from jax.experimental import pallas as pl
from jax.experimental.pallas import tpu as pltpu
from jax.experimental.pallas import tpu_sc as plsc

_N_E = 8192
_D = 32
_BETA = 0.25
_B = 8 * 576
_BM = 256
_G = _B // _BM

_NC = 2
_NS = 16
_NW = _NC * _NS
_BPW = _B // _NW


def _main_body(z_ref, embt_ref, zn_ref, idx_ref, ap_ref, ent_ref, hist_ref,
               embn_ref, lesq_ref):
    i = pl.program_id(0)

    @pl.when(i == 0)
    def _init():
        embt = embt_ref[...]
        nsq = jnp.sum(embt * embt, axis=0, keepdims=True)
        embn = embt / jnp.maximum(jnp.sqrt(nsq), 1e-12)
        embn_ref[...] = embn
        lesq_ref[...] = 10.0 * jnp.sum(embn * embn, axis=0, keepdims=True) - 30.0
        ap_ref[...] = jnp.zeros_like(ap_ref)
        ent_ref[...] = jnp.zeros_like(ent_ref)
        hist_ref[...] = jnp.zeros_like(hist_ref)

    z = z_ref[...]
    zn = z / jnp.maximum(jnp.sqrt(jnp.sum(z * z, axis=1, keepdims=True)), 1e-12)
    zn_ref[...] = zn

    dot = jnp.dot(zn, embn_ref[...], preferred_element_type=jnp.float32)
    lc = lesq_ref[...] - 20.0 * dot
    e = jnp.exp(lc)
    lmin = jnp.min(lc, axis=1, keepdims=True)

    s = jnp.sum(e, axis=1, keepdims=True)
    inv = 1.0 / s
    p = e * inv
    ap_ref[...] += jnp.sum(p, axis=0, keepdims=True)
    plc = p * lc
    rowent = jnp.log(s) - jnp.sum(plc, axis=1, keepdims=True)
    ent_ref[...] += jnp.sum(rowent).reshape(1, 1)

    cols = lax.broadcasted_iota(jnp.int32, (_BM, _N_E), 1)
    idx = jnp.min(jnp.where(lc == lmin, cols, _N_E), axis=1)
    idx_ref[...] = idx.reshape(1, 1, _BM)
    onehot = jnp.where(cols == idx.reshape(_BM, 1), 1.0, 0.0)
    hist_ref[...] += jnp.sum(onehot, axis=0, keepdims=True)


def _main_tc(z_flat, embt):
    return pl.pallas_call(
        _main_body,
        grid=(_G,),
        in_specs=[
            pl.BlockSpec((_BM, _D), lambda i: (i, 0)),
            pl.BlockSpec((_D, _N_E), lambda i: (0, 0)),
        ],
        out_specs=[
            pl.BlockSpec((_BM, _D), lambda i: (i, 0)),
            pl.BlockSpec((1, 1, _BM), lambda i: (i, 0, 0)),
            pl.BlockSpec((1, _N_E), lambda i: (0, 0)),
            pl.BlockSpec((1, 1), lambda i: (0, 0)),
            pl.BlockSpec((1, _N_E), lambda i: (0, 0)),
        ],
        out_shape=[
            jax.ShapeDtypeStruct((_B, _D), jnp.float32),
            jax.ShapeDtypeStruct((_G, 1, _BM), jnp.int32),
            jax.ShapeDtypeStruct((1, _N_E), jnp.float32),
            jax.ShapeDtypeStruct((1, 1), jnp.float32),
            jax.ShapeDtypeStruct((1, _N_E), jnp.float32),
        ],
        scratch_shapes=[
            pltpu.VMEM((_D, _N_E), jnp.float32),
            pltpu.VMEM((1, _N_E), jnp.float32),
        ],
        compiler_params=pltpu.CompilerParams(
            dimension_semantics=("arbitrary",)),
    )(z_flat, embt)


def _sc_gather_body(emb_hbm, idx_hbm, out_hbm, idx_v, rows_v, sem):
    wid = lax.axis_index("s") * _NC + lax.axis_index("c")
    base = wid * _BPW
    pltpu.sync_copy(idx_hbm.at[pl.ds(base, _BPW)], idx_v)
    pltpu.async_copy(emb_hbm.at[idx_v], rows_v, sem).wait()
    pltpu.sync_copy(rows_v, out_hbm.at[pl.ds(base, _BPW)])


def _sc_gather(emb, idx):
    mesh = plsc.VectorSubcoreMesh(core_axis_name="c", subcore_axis_name="s")
    f = pl.kernel(
        _sc_gather_body,
        mesh=mesh,
        out_type=jax.ShapeDtypeStruct((_B, _D), jnp.float32),
        scratch_types=[
            pltpu.VMEM((_BPW,), jnp.int32),
            pltpu.VMEM((_BPW, _D), jnp.float32),
            pltpu.SemaphoreType.DMA,
        ],
        compiler_params=pltpu.CompilerParams(use_tc_tiling_on_sc=False),
    )
    return f(emb, idx)


def _fin_body(zn_ref, zq_ref, ap_ref, ent_ref, hist_ref,
              out_ref, loss_ref, perp_ref, entmin_ref):
    zq = zq_ref[...]
    zqn = zq / jnp.maximum(jnp.sqrt(jnp.sum(zq * zq, axis=1, keepdims=True)),
                           1e-12)
    zn = zn_ref[...]
    out_ref[...] = zn + (zqn - zn)
    diff = zqn - zn
    commit = (1.0 + _BETA) * jnp.mean(diff * diff)
    ap = ap_ref[...] * (1.0 / _B)
    ent_max = -jnp.sum(ap * jnp.log(ap))
    loss_ref[...] = (commit - ent_max).reshape(1, 1)
    probs = hist_ref[...] * (1.0 / _B)
    perp_ref[...] = jnp.exp(-jnp.sum(probs * jnp.log(probs + 1e-10))).reshape(1, 1)
    entmin_ref[...] = ent_ref[...] * (1.0 / _B)


def _finalize(zn, zq, ap_sum, ent_sum, hist):
    return pl.pallas_call(
        _fin_body,
        out_shape=[
            jax.ShapeDtypeStruct((_B, _D), jnp.float32),
            jax.ShapeDtypeStruct((1, 1), jnp.float32),
            jax.ShapeDtypeStruct((1, 1), jnp.float32),
            jax.ShapeDtypeStruct((1, 1), jnp.float32),
        ],
    )(zn, zq, ap_sum, ent_sum, hist)


def kernel(z, emb):
    z_flat = z.reshape(_B, _D)
    embt = emb.T
    zn, idx3, ap_sum, ent_sum, hist = _main_tc(z_flat, embt)
    idx = idx3.reshape(_B)
    zq = _sc_gather(emb, idx)
    zq_out, loss, perp, entmin = _finalize(zn, zq, ap_sum, ent_sum, hist)
    return (zq_out.reshape(z.shape), idx, loss[0, 0], perp[0, 0],
            entmin[0, 0])


# speedup vs baseline: 1.1691x; 1.0015x over previous
import jax
import jax.numpy as jnp
from jax import lax
from jax.experimental import pallas as pl
from jax.experimental.pallas import tpu as pltpu
from jax.experimental.pallas import tpu_sc as plsc

_N_E = 8192
_D = 32
_BETA = 0.25
_B = 8 * 576
_BM = 256
_G = _B // _BM

_NC = 2
_NS = 16
_NW = _NC * _NS
_BPW = _B // _NW


def _main_body(z_ref, emb_ref, idx_ref, ap_ref, ent_ref, hist_ref,
               embn_ref, lesq_ref):
    i = pl.program_id(0)

    @pl.when(i == 0)
    def _init():
        emb = emb_ref[...]
        nsq = jnp.sum(emb * emb, axis=1, keepdims=True)
        embn = emb / jnp.maximum(jnp.sqrt(nsq), 1e-12)
        embn_ref[...] = embn
        lesq_ref[...] = (10.0 * jnp.sum(embn * embn, axis=1) - 30.0).reshape(1, _N_E)
        ap_ref[...] = jnp.zeros_like(ap_ref)
        ent_ref[...] = jnp.zeros_like(ent_ref)
        hist_ref[...] = jnp.zeros_like(hist_ref)

    z = z_ref[...]
    zn = z / jnp.maximum(jnp.sqrt(jnp.sum(z * z, axis=1, keepdims=True)), 1e-12)

    dot = lax.dot_general(zn, embn_ref[...], (((1,), (1,)), ((), ())),
                          preferred_element_type=jnp.float32)
    lc = lesq_ref[...] - 20.0 * dot
    e = jnp.exp(lc)
    s = jnp.sum(e, axis=1, keepdims=True)
    inv = 1.0 / s
    p = e * inv
    ap_ref[...] += jnp.sum(p, axis=0, keepdims=True)
    plc = p * lc
    rowent = jnp.log(s) - jnp.sum(plc, axis=1, keepdims=True)
    ent_ref[...] += jnp.sum(rowent).reshape(1, 1)

    idx = jnp.argmin(lc, axis=1).astype(jnp.int32)
    idx_ref[...] = idx.reshape(1, 1, _BM)
    cols = lax.broadcasted_iota(jnp.int32, (_BM, _N_E), 1)
    onehot = jnp.where(cols == idx.reshape(_BM, 1), 1.0, 0.0)
    hist_ref[...] += jnp.sum(onehot, axis=0, keepdims=True)


def _main_tc(z_flat, emb):
    return pl.pallas_call(
        _main_body,
        grid=(_G,),
        in_specs=[
            pl.BlockSpec((_BM, _D), lambda i: (i, 0)),
            pl.BlockSpec((_N_E, _D), lambda i: (0, 0)),
        ],
        out_specs=[
            pl.BlockSpec((1, 1, _BM), lambda i: (i, 0, 0)),
            pl.BlockSpec((1, _N_E), lambda i: (0, 0)),
            pl.BlockSpec((1, 1), lambda i: (0, 0)),
            pl.BlockSpec((1, _N_E), lambda i: (0, 0)),
        ],
        out_shape=[
            jax.ShapeDtypeStruct((_G, 1, _BM), jnp.int32),
            jax.ShapeDtypeStruct((1, _N_E), jnp.float32),
            jax.ShapeDtypeStruct((1, 1), jnp.float32),
            jax.ShapeDtypeStruct((1, _N_E), jnp.float32),
        ],
        scratch_shapes=[
            pltpu.VMEM((_N_E, _D), jnp.float32),
            pltpu.VMEM((1, _N_E), jnp.float32),
        ],
        compiler_params=pltpu.CompilerParams(
            dimension_semantics=("arbitrary",)),
    )(z_flat, emb)


def _sc_gather_body(emb_hbm, idx_hbm, out_hbm, idx_v, rows_v, sem):
    wid = lax.axis_index("s") * _NC + lax.axis_index("c")
    base = wid * _BPW
    pltpu.sync_copy(idx_hbm.at[pl.ds(base, _BPW)], idx_v)
    pltpu.async_copy(emb_hbm.at[idx_v], rows_v, sem).wait()
    pltpu.sync_copy(rows_v, out_hbm.at[pl.ds(base, _BPW)])


def _sc_gather(emb, idx):
    mesh = plsc.VectorSubcoreMesh(core_axis_name="c", subcore_axis_name="s")
    f = pl.kernel(
        _sc_gather_body,
        mesh=mesh,
        out_type=jax.ShapeDtypeStruct((_B, _D), jnp.float32),
        scratch_types=[
            pltpu.VMEM((_BPW,), jnp.int32),
            pltpu.VMEM((_BPW, _D), jnp.float32),
            pltpu.SemaphoreType.DMA,
        ],
        compiler_params=pltpu.CompilerParams(use_tc_tiling_on_sc=False),
    )
    return f(emb, idx)


def _fin_body(z_ref, zq_ref, ap_ref, ent_ref, hist_ref,
              out_ref, loss_ref, perp_ref, entmin_ref):
    z = z_ref[...]
    zn = z / jnp.maximum(jnp.sqrt(jnp.sum(z * z, axis=1, keepdims=True)), 1e-12)
    zq = zq_ref[...]
    zqn = zq / jnp.maximum(jnp.sqrt(jnp.sum(zq * zq, axis=1, keepdims=True)),
                           1e-12)
    out_ref[...] = zn + (zqn - zn)
    diff = zqn - zn
    commit = (1.0 + _BETA) * jnp.mean(diff * diff)
    ap = ap_ref[...] * (1.0 / _B)
    ent_max = -jnp.sum(ap * jnp.log(ap))
    loss_ref[...] = (commit - ent_max).reshape(1, 1)
    probs = hist_ref[...] * (1.0 / _B)
    perp_ref[...] = jnp.exp(-jnp.sum(probs * jnp.log(probs + 1e-10))).reshape(1, 1)
    entmin_ref[...] = ent_ref[...] * (1.0 / _B)


def _finalize(z_flat, zq, ap_sum, ent_sum, hist):
    return pl.pallas_call(
        _fin_body,
        out_shape=[
            jax.ShapeDtypeStruct((_B, _D), jnp.float32),
            jax.ShapeDtypeStruct((1, 1), jnp.float32),
            jax.ShapeDtypeStruct((1, 1), jnp.float32),
            jax.ShapeDtypeStruct((1, 1), jnp.float32),
        ],
    )(z_flat, zq, ap_sum, ent_sum, hist)


def kernel(z, emb):
    z_flat = z.reshape(_B, _D)
    idx3, ap_sum, ent_sum, hist = _main_tc(z_flat, emb)
    idx = idx3.reshape(_B)
    zq = _sc_gather(emb, idx)
    zq_out, loss, perp, entmin = _finalize(z_flat, zq, ap_sum, ent_sum, hist)
    return (zq_out.reshape(z.shape), idx, loss[0, 0], perp[0, 0],
            entmin[0, 0])


# speedup vs baseline: 1.2190x; 1.0427x over previous
import jax
import jax.numpy as jnp
from jax import lax
from jax.experimental import pallas as pl
from jax.experimental.pallas import tpu as pltpu
from jax.experimental.pallas import tpu_sc as plsc

_N_E = 8192
_D = 32
_BETA = 0.25
_B = 8 * 576
_BM = 256
_G = _B // _BM

_NC = 2
_NS = 16
_NW = _NC * _NS
_BPW = _B // _NW


def _main_body(z_ref, embt_ref, idx_ref, ap_ref, ent_ref,
               embn_ref, esq_ref):
    i = pl.program_id(0)

    @pl.when(i == 0)
    def _init():
        embt = embt_ref[...]
        nsq = jnp.sum(embt * embt, axis=0, keepdims=True)
        embn = embt / jnp.maximum(jnp.sqrt(nsq), 1e-12)
        embn_ref[...] = embn
        esq_ref[...] = jnp.sum(embn * embn, axis=0, keepdims=True)
        ap_ref[...] = jnp.zeros_like(ap_ref)
        ent_ref[...] = jnp.zeros_like(ent_ref)

    z = z_ref[...]
    zn = z / jnp.maximum(jnp.sqrt(jnp.sum(z * z, axis=1, keepdims=True)), 1e-12)
    zsq = jnp.sum(zn * zn, axis=1, keepdims=True)

    dot = jnp.dot(zn, embn_ref[...], preferred_element_type=jnp.float32)
    d = (zsq + esq_ref[...]) - 2.0 * dot
    idx = jnp.argmin(d, axis=1).astype(jnp.int32)
    idx_ref[...] = idx.reshape(1, 1, _BM)

    lc = 10.0 * d - 40.0
    e = jnp.exp(lc)
    s = jnp.sum(e, axis=1, keepdims=True)
    inv = 1.0 / s
    p = e * inv
    ap_ref[...] += jnp.sum(p, axis=0, keepdims=True)
    plc = p * lc
    rowent = jnp.log(s) - jnp.sum(plc, axis=1, keepdims=True)
    ent_ref[...] += jnp.sum(rowent).reshape(1, 1)


def _main_tc(z_flat, embt):
    return pl.pallas_call(
        _main_body,
        grid=(_G,),
        in_specs=[
            pl.BlockSpec((_BM, _D), lambda i: (i, 0)),
            pl.BlockSpec((_D, _N_E), lambda i: (0, 0)),
        ],
        out_specs=[
            pl.BlockSpec((1, 1, _BM), lambda i: (i, 0, 0)),
            pl.BlockSpec((1, _N_E), lambda i: (0, 0)),
            pl.BlockSpec((1, 1), lambda i: (0, 0)),
        ],
        out_shape=[
            jax.ShapeDtypeStruct((_G, 1, _BM), jnp.int32),
            jax.ShapeDtypeStruct((1, _N_E), jnp.float32),
            jax.ShapeDtypeStruct((1, 1), jnp.float32),
        ],
        scratch_shapes=[
            pltpu.VMEM((_D, _N_E), jnp.float32),
            pltpu.VMEM((1, _N_E), jnp.float32),
        ],
        compiler_params=pltpu.CompilerParams(
            dimension_semantics=("arbitrary",)),
    )(z_flat, embt)


def _sc_gather_body(emb_hbm, idx_hbm, out_hbm, hist_hbm, idx_v, rows_v,
                    hist_v, sem):
    wid = lax.axis_index("s") * _NC + lax.axis_index("c")
    base = wid * _BPW
    pltpu.sync_copy(idx_hbm.at[pl.ds(base, _BPW)], idx_v)
    copy = pltpu.async_copy(emb_hbm.at[idx_v], rows_v, sem)

    def _zero(k, _):
        hist_v[pl.ds(k * 16, 16)] = jnp.zeros((16,), jnp.float32)
        return _
    lax.fori_loop(0, _N_E // 16, _zero, 0)
    ones = jnp.ones((16,), jnp.float32)
    for j in range(_BPW // 16):
        plsc.addupdate_scatter(hist_v, [idx_v[pl.ds(j * 16, 16)]], ones)
    pltpu.sync_copy(hist_v, hist_hbm.at[wid])

    copy.wait()
    pltpu.sync_copy(rows_v, out_hbm.at[pl.ds(base, _BPW)])


def _sc_gather(emb, idx):
    mesh = plsc.VectorSubcoreMesh(core_axis_name="c", subcore_axis_name="s")
    f = pl.kernel(
        _sc_gather_body,
        mesh=mesh,
        out_type=[
            jax.ShapeDtypeStruct((_B, _D), jnp.float32),
            jax.ShapeDtypeStruct((_NW, _N_E), jnp.float32),
        ],
        scratch_types=[
            pltpu.VMEM((_BPW,), jnp.int32),
            pltpu.VMEM((_BPW, _D), jnp.float32),
            pltpu.VMEM((_N_E,), jnp.float32),
            pltpu.SemaphoreType.DMA,
        ],
        compiler_params=pltpu.CompilerParams(use_tc_tiling_on_sc=False,
                                             needs_layout_passes=False),
    )
    return f(emb, idx)


def _fin_body(z_ref, zq_ref, ap_ref, ent_ref, histsc_ref,
              out_ref, loss_ref, perp_ref, entmin_ref):
    z = z_ref[...]
    zn = z / jnp.maximum(jnp.sqrt(jnp.sum(z * z, axis=1, keepdims=True)), 1e-12)
    zq = zq_ref[...]
    zqn = zq / jnp.maximum(jnp.sqrt(jnp.sum(zq * zq, axis=1, keepdims=True)),
                           1e-12)
    out_ref[...] = zn + (zqn - zn)
    diff = zqn - zn
    commit = (1.0 + _BETA) * jnp.mean(diff * diff)
    ap = ap_ref[...] * (1.0 / _B)
    ent_max = -jnp.sum(ap * jnp.log(ap))
    hist = jnp.sum(histsc_ref[...], axis=0, keepdims=True)
    loss_ref[...] = (commit - ent_max).reshape(1, 1)
    probs = hist * (1.0 / _B)
    perp_ref[...] = jnp.exp(-jnp.sum(probs * jnp.log(probs + 1e-10))).reshape(1, 1)
    entmin_ref[...] = ent_ref[...] * (1.0 / _B)


def _finalize(z_flat, zq, ap_sum, ent_sum, hist_sc):
    return pl.pallas_call(
        _fin_body,
        out_shape=[
            jax.ShapeDtypeStruct((_B, _D), jnp.float32),
            jax.ShapeDtypeStruct((1, 1), jnp.float32),
            jax.ShapeDtypeStruct((1, 1), jnp.float32),
            jax.ShapeDtypeStruct((1, 1), jnp.float32),
        ],
    )(z_flat, zq, ap_sum, ent_sum, hist_sc)


def kernel(z, emb):
    z_flat = z.reshape(_B, _D)
    idx3, ap_sum, ent_sum = _main_tc(z_flat, emb.T)
    idx = idx3.reshape(_B)
    zq, hist_sc = _sc_gather(emb, idx)
    zq_out, loss, perp, entmin = _finalize(z_flat, zq, ap_sum, ent_sum,
                                           hist_sc)
    return (zq_out.reshape(z.shape), idx, loss[0, 0], perp[0, 0],
            entmin[0, 0])


# speedup vs baseline: 1.2527x; 1.0277x over previous
import jax
import jax.numpy as jnp
from jax import lax
from jax.experimental import pallas as pl
from jax.experimental.pallas import tpu as pltpu
from jax.experimental.pallas import tpu_sc as plsc

_N_E = 8192
_D = 32
_BETA = 0.25
_B = 8 * 576
_BM = 512
_G = _B // _BM

_NC = 2
_NS = 16
_NW = _NC * _NS
_BPW = _B // _NW


def _main_body(z_ref, embt_ref, idx_ref, ap_ref, ent_ref,
               embn_ref, esq_ref):
    i = pl.program_id(0)

    @pl.when(i == 0)
    def _init():
        embt = embt_ref[...]
        nsq = jnp.sum(embt * embt, axis=0, keepdims=True)
        embn = embt / jnp.maximum(jnp.sqrt(nsq), 1e-12)
        embn_ref[...] = embn
        esq_ref[...] = jnp.sum(embn * embn, axis=0, keepdims=True)
        ap_ref[...] = jnp.zeros_like(ap_ref)
        ent_ref[...] = jnp.zeros_like(ent_ref)

    z = z_ref[...]
    zn = z / jnp.maximum(jnp.sqrt(jnp.sum(z * z, axis=1, keepdims=True)), 1e-12)
    zsq = jnp.sum(zn * zn, axis=1, keepdims=True)

    dot = jnp.dot(zn, embn_ref[...], preferred_element_type=jnp.float32)
    d = (zsq + esq_ref[...]) - 2.0 * dot
    idx = jnp.argmin(d, axis=1).astype(jnp.int32)
    idx_ref[...] = idx.reshape(1, 1, _BM)

    lc = 10.0 * d - 40.0
    e = jnp.exp(lc)
    s = jnp.sum(e, axis=1, keepdims=True)
    inv = 1.0 / s
    p = e * inv
    ap_ref[...] += jnp.sum(p, axis=0, keepdims=True)
    plc = p * lc
    rowent = jnp.log(s) - jnp.sum(plc, axis=1, keepdims=True)
    ent_ref[...] += jnp.sum(rowent).reshape(1, 1)


def _main_tc(z_flat, embt):
    return pl.pallas_call(
        _main_body,
        grid=(_G,),
        in_specs=[
            pl.BlockSpec((_BM, _D), lambda i: (i, 0)),
            pl.BlockSpec((_D, _N_E), lambda i: (0, 0)),
        ],
        out_specs=[
            pl.BlockSpec((1, 1, _BM), lambda i: (i, 0, 0)),
            pl.BlockSpec((1, _N_E), lambda i: (0, 0)),
            pl.BlockSpec((1, 1), lambda i: (0, 0)),
        ],
        out_shape=[
            jax.ShapeDtypeStruct((_G, 1, _BM), jnp.int32),
            jax.ShapeDtypeStruct((1, _N_E), jnp.float32),
            jax.ShapeDtypeStruct((1, 1), jnp.float32),
        ],
        scratch_shapes=[
            pltpu.VMEM((_D, _N_E), jnp.float32),
            pltpu.VMEM((1, _N_E), jnp.float32),
        ],
        compiler_params=pltpu.CompilerParams(
            dimension_semantics=("arbitrary",)),
    )(z_flat, embt)


def _sc_gather_body(emb_hbm, idx_hbm, out_hbm, hist_hbm, idx_v, rows_v,
                    hist_v, sem):
    wid = lax.axis_index("s") * _NC + lax.axis_index("c")
    base = wid * _BPW
    pltpu.sync_copy(idx_hbm.at[pl.ds(base, _BPW)], idx_v)
    copy = pltpu.async_copy(emb_hbm.at[idx_v], rows_v, sem)

    def _zero(k, _):
        hist_v[pl.ds(k * 16, 16)] = jnp.zeros((16,), jnp.float32)
        return _
    lax.fori_loop(0, _N_E // 16, _zero, 0)
    ones = jnp.ones((16,), jnp.float32)
    for j in range(_BPW // 16):
        plsc.addupdate_scatter(hist_v, [idx_v[pl.ds(j * 16, 16)]], ones)
    pltpu.sync_copy(hist_v, hist_hbm.at[wid])

    copy.wait()
    pltpu.sync_copy(rows_v, out_hbm.at[pl.ds(base, _BPW)])


def _sc_gather(emb, idx):
    mesh = plsc.VectorSubcoreMesh(core_axis_name="c", subcore_axis_name="s")
    f = pl.kernel(
        _sc_gather_body,
        mesh=mesh,
        out_type=[
            jax.ShapeDtypeStruct((_B, _D), jnp.float32),
            jax.ShapeDtypeStruct((_NW, _N_E), jnp.float32),
        ],
        scratch_types=[
            pltpu.VMEM((_BPW,), jnp.int32),
            pltpu.VMEM((_BPW, _D), jnp.float32),
            pltpu.VMEM((_N_E,), jnp.float32),
            pltpu.SemaphoreType.DMA,
        ],
        compiler_params=pltpu.CompilerParams(use_tc_tiling_on_sc=False,
                                             needs_layout_passes=False),
    )
    return f(emb, idx)


def _fin_body(z_ref, zq_ref, ap_ref, ent_ref, histsc_ref,
              out_ref, loss_ref, perp_ref, entmin_ref):
    z = z_ref[...]
    zn = z / jnp.maximum(jnp.sqrt(jnp.sum(z * z, axis=1, keepdims=True)), 1e-12)
    zq = zq_ref[...]
    zqn = zq / jnp.maximum(jnp.sqrt(jnp.sum(zq * zq, axis=1, keepdims=True)),
                           1e-12)
    out_ref[...] = zn + (zqn - zn)
    diff = zqn - zn
    commit = (1.0 + _BETA) * jnp.mean(diff * diff)
    ap = ap_ref[...] * (1.0 / _B)
    ent_max = -jnp.sum(ap * jnp.log(ap))
    hist = jnp.sum(histsc_ref[...], axis=0, keepdims=True)
    loss_ref[...] = (commit - ent_max).reshape(1, 1)
    probs = hist * (1.0 / _B)
    perp_ref[...] = jnp.exp(-jnp.sum(probs * jnp.log(probs + 1e-10))).reshape(1, 1)
    entmin_ref[...] = ent_ref[...] * (1.0 / _B)


def _finalize(z_flat, zq, ap_sum, ent_sum, hist_sc):
    return pl.pallas_call(
        _fin_body,
        out_shape=[
            jax.ShapeDtypeStruct((_B, _D), jnp.float32),
            jax.ShapeDtypeStruct((1, 1), jnp.float32),
            jax.ShapeDtypeStruct((1, 1), jnp.float32),
            jax.ShapeDtypeStruct((1, 1), jnp.float32),
        ],
    )(z_flat, zq, ap_sum, ent_sum, hist_sc)


def kernel(z, emb):
    z_flat = z.reshape(_B, _D)
    idx3, ap_sum, ent_sum = _main_tc(z_flat, emb.T)
    idx = idx3.reshape(_B)
    zq, hist_sc = _sc_gather(emb, idx)
    zq_out, loss, perp, entmin = _finalize(z_flat, zq, ap_sum, ent_sum,
                                           hist_sc)
    return (zq_out.reshape(z.shape), idx, loss[0, 0], perp[0, 0],
            entmin[0, 0])


# speedup vs baseline: 1.2638x; 1.0088x over previous
import jax
import jax.numpy as jnp
from jax import lax
from jax.experimental import pallas as pl
from jax.experimental.pallas import tpu as pltpu
from jax.experimental.pallas import tpu_sc as plsc

_N_E = 8192
_D = 32
_BETA = 0.25
_B = 8 * 576
_BM = 576
_G = _B // _BM

_NC = 2
_NS = 16
_NW = _NC * _NS
_BPW = _B // _NW


def _main_body(z_ref, embt_ref, idx_ref, ap_ref, ent_ref,
               embn_ref, esq_ref):
    i = pl.program_id(0)

    @pl.when(i == 0)
    def _init():
        embt = embt_ref[...]
        nsq = jnp.sum(embt * embt, axis=0, keepdims=True)
        embn = embt / jnp.maximum(jnp.sqrt(nsq), 1e-12)
        embn_ref[...] = embn
        esq_ref[...] = jnp.sum(embn * embn, axis=0, keepdims=True)
        ap_ref[...] = jnp.zeros_like(ap_ref)
        ent_ref[...] = jnp.zeros_like(ent_ref)

    z = z_ref[...]
    zn = z / jnp.maximum(jnp.sqrt(jnp.sum(z * z, axis=1, keepdims=True)), 1e-12)
    zsq = jnp.sum(zn * zn, axis=1, keepdims=True)

    dot = jnp.dot(zn, embn_ref[...], preferred_element_type=jnp.float32)
    d = (zsq + esq_ref[...]) - 2.0 * dot
    idx = jnp.argmin(d, axis=1).astype(jnp.int32)
    idx_ref[...] = idx.reshape(1, 1, _BM)

    lc = 10.0 * d - 40.0
    e = jnp.exp(lc)
    s = jnp.sum(e, axis=1, keepdims=True)
    inv = 1.0 / s
    p = e * inv
    ap_ref[...] += jnp.sum(p, axis=0, keepdims=True)
    plc = p * lc
    rowent = jnp.log(s) - jnp.sum(plc, axis=1, keepdims=True)
    ent_ref[...] += jnp.sum(rowent).reshape(1, 1)


def _main_tc(z_flat, embt):
    return pl.pallas_call(
        _main_body,
        grid=(_G,),
        in_specs=[
            pl.BlockSpec((_BM, _D), lambda i: (i, 0)),
            pl.BlockSpec((_D, _N_E), lambda i: (0, 0)),
        ],
        out_specs=[
            pl.BlockSpec((1, 1, _BM), lambda i: (i, 0, 0)),
            pl.BlockSpec((1, _N_E), lambda i: (0, 0)),
            pl.BlockSpec((1, 1), lambda i: (0, 0)),
        ],
        out_shape=[
            jax.ShapeDtypeStruct((_G, 1, _BM), jnp.int32),
            jax.ShapeDtypeStruct((1, _N_E), jnp.float32),
            jax.ShapeDtypeStruct((1, 1), jnp.float32),
        ],
        scratch_shapes=[
            pltpu.VMEM((_D, _N_E), jnp.float32),
            pltpu.VMEM((1, _N_E), jnp.float32),
        ],
        compiler_params=pltpu.CompilerParams(
            dimension_semantics=("arbitrary",)),
    )(z_flat, embt)


def _sc_gather_body(emb_hbm, idx_hbm, out_hbm, hist_hbm, idx_v, rows_v,
                    hist_v, sem):
    wid = lax.axis_index("s") * _NC + lax.axis_index("c")
    base = wid * _BPW
    pltpu.sync_copy(idx_hbm.at[pl.ds(base, _BPW)], idx_v)
    copy = pltpu.async_copy(emb_hbm.at[idx_v], rows_v, sem)

    def _zero(k, _):
        hist_v[pl.ds(k * 16, 16)] = jnp.zeros((16,), jnp.float32)
        return _
    lax.fori_loop(0, _N_E // 16, _zero, 0)
    ones = jnp.ones((16,), jnp.float32)
    for j in range(_BPW // 16):
        plsc.addupdate_scatter(hist_v, [idx_v[pl.ds(j * 16, 16)]], ones)
    pltpu.sync_copy(hist_v, hist_hbm.at[wid])

    copy.wait()
    pltpu.sync_copy(rows_v, out_hbm.at[pl.ds(base, _BPW)])


def _sc_gather(emb, idx):
    mesh = plsc.VectorSubcoreMesh(core_axis_name="c", subcore_axis_name="s")
    f = pl.kernel(
        _sc_gather_body,
        mesh=mesh,
        out_type=[
            jax.ShapeDtypeStruct((_B, _D), jnp.float32),
            jax.ShapeDtypeStruct((_NW, _N_E), jnp.float32),
        ],
        scratch_types=[
            pltpu.VMEM((_BPW,), jnp.int32),
            pltpu.VMEM((_BPW, _D), jnp.float32),
            pltpu.VMEM((_N_E,), jnp.float32),
            pltpu.SemaphoreType.DMA,
        ],
        compiler_params=pltpu.CompilerParams(use_tc_tiling_on_sc=False,
                                             needs_layout_passes=False),
    )
    return f(emb, idx)


def _fin_body(z_ref, zq_ref, ap_ref, ent_ref, histsc_ref,
              out_ref, loss_ref, perp_ref, entmin_ref):
    z = z_ref[...]
    zn = z / jnp.maximum(jnp.sqrt(jnp.sum(z * z, axis=1, keepdims=True)), 1e-12)
    zq = zq_ref[...]
    zqn = zq / jnp.maximum(jnp.sqrt(jnp.sum(zq * zq, axis=1, keepdims=True)),
                           1e-12)
    out_ref[...] = zn + (zqn - zn)
    diff = zqn - zn
    commit = (1.0 + _BETA) * jnp.mean(diff * diff)
    ap = ap_ref[...] * (1.0 / _B)
    ent_max = -jnp.sum(ap * jnp.log(ap))
    hist = jnp.sum(histsc_ref[...], axis=0, keepdims=True)
    loss_ref[...] = (commit - ent_max).reshape(1, 1)
    probs = hist * (1.0 / _B)
    perp_ref[...] = jnp.exp(-jnp.sum(probs * jnp.log(probs + 1e-10))).reshape(1, 1)
    entmin_ref[...] = ent_ref[...] * (1.0 / _B)


def _finalize(z_flat, zq, ap_sum, ent_sum, hist_sc):
    return pl.pallas_call(
        _fin_body,
        out_shape=[
            jax.ShapeDtypeStruct((_B, _D), jnp.float32),
            jax.ShapeDtypeStruct((1, 1), jnp.float32),
            jax.ShapeDtypeStruct((1, 1), jnp.float32),
            jax.ShapeDtypeStruct((1, 1), jnp.float32),
        ],
    )(z_flat, zq, ap_sum, ent_sum, hist_sc)


def kernel(z, emb):
    z_flat = z.reshape(_B, _D)
    idx3, ap_sum, ent_sum = _main_tc(z_flat, emb.T)
    idx = idx3.reshape(_B)
    zq, hist_sc = _sc_gather(emb, idx)
    zq_out, loss, perp, entmin = _finalize(z_flat, zq, ap_sum, ent_sum,
                                           hist_sc)
    return (zq_out.reshape(z.shape), idx, loss[0, 0], perp[0, 0],
            entmin[0, 0])


# speedup vs baseline: 1.3528x; 1.0704x over previous
import jax
import jax.numpy as jnp
from jax import lax
from jax.experimental import pallas as pl
from jax.experimental.pallas import tpu as pltpu
from jax.experimental.pallas import tpu_sc as plsc

_N_E = 8192
_D = 32
_BETA = 0.25
_B = 8 * 576
_BM = 576
_G = _B // _BM

_NC = 2
_NS = 16
_NW = _NC * _NS
_BPW = _B // _NW


def _main_body(z_ref, embt_ref, idx_ref, ap_ref, ent_ref,
               embn_ref, esq_ref):
    i = pl.program_id(0)

    @pl.when(i == 0)
    def _init():
        embt = embt_ref[...]
        nsq = jnp.sum(embt * embt, axis=0, keepdims=True)
        embn = embt / jnp.maximum(jnp.sqrt(nsq), 1e-12)
        embn_ref[...] = embn
        esq_ref[...] = jnp.sum(embn * embn, axis=0, keepdims=True)
        ap_ref[...] = jnp.zeros_like(ap_ref)
        ent_ref[...] = jnp.zeros_like(ent_ref)

    z = z_ref[...]
    zn = z / jnp.maximum(jnp.sqrt(jnp.sum(z * z, axis=1, keepdims=True)), 1e-12)
    zsq = jnp.sum(zn * zn, axis=1, keepdims=True)

    dot = jnp.dot(zn, embn_ref[...], preferred_element_type=jnp.float32)
    d = (zsq + esq_ref[...]) - 2.0 * dot
    idx = jnp.argmin(d, axis=1).astype(jnp.int32)
    idx_ref[...] = idx.reshape(1, 1, _BM)

    lc = 10.0 * d - 40.0
    e = jnp.exp(lc)
    s = jnp.sum(e, axis=1, keepdims=True)
    inv = 1.0 / s
    ap_ref[...] += jnp.sum(e * inv, axis=0, keepdims=True)
    rowent = jnp.log(s) - inv * jnp.sum(e * lc, axis=1, keepdims=True)
    ent_ref[...] += jnp.sum(rowent).reshape(1, 1)


def _main_tc(z_flat, embt):
    return pl.pallas_call(
        _main_body,
        grid=(_G,),
        in_specs=[
            pl.BlockSpec((_BM, _D), lambda i: (i, 0)),
            pl.BlockSpec((_D, _N_E), lambda i: (0, 0)),
        ],
        out_specs=[
            pl.BlockSpec((1, 1, _BM), lambda i: (i, 0, 0)),
            pl.BlockSpec((1, _N_E), lambda i: (0, 0)),
            pl.BlockSpec((1, 1), lambda i: (0, 0)),
        ],
        out_shape=[
            jax.ShapeDtypeStruct((_G, 1, _BM), jnp.int32),
            jax.ShapeDtypeStruct((1, _N_E), jnp.float32),
            jax.ShapeDtypeStruct((1, 1), jnp.float32),
        ],
        scratch_shapes=[
            pltpu.VMEM((_D, _N_E), jnp.float32),
            pltpu.VMEM((1, _N_E), jnp.float32),
        ],
        compiler_params=pltpu.CompilerParams(
            dimension_semantics=("arbitrary",)),
    )(z_flat, embt)


def _sc_gather_body(emb_hbm, idx_hbm, out_hbm, hist_hbm, idx_v, rows_v,
                    hist_v, sem):
    wid = lax.axis_index("s") * _NC + lax.axis_index("c")
    base = wid * _BPW
    pltpu.sync_copy(idx_hbm.at[pl.ds(base, _BPW)], idx_v)
    copy = pltpu.async_copy(emb_hbm.at[idx_v], rows_v, sem)

    def _zero(k, _):
        hist_v[pl.ds(k * 16, 16)] = jnp.zeros((16,), jnp.float32)
        return _
    lax.fori_loop(0, _N_E // 16, _zero, 0)
    ones = jnp.ones((16,), jnp.float32)
    for j in range(_BPW // 16):
        plsc.addupdate_scatter(hist_v, [idx_v[pl.ds(j * 16, 16)]], ones)
    pltpu.sync_copy(hist_v, hist_hbm.at[wid])

    copy.wait()
    pltpu.sync_copy(rows_v, out_hbm.at[pl.ds(base, _BPW)])


def _sc_gather(emb, idx):
    mesh = plsc.VectorSubcoreMesh(core_axis_name="c", subcore_axis_name="s")
    f = pl.kernel(
        _sc_gather_body,
        mesh=mesh,
        out_type=[
            jax.ShapeDtypeStruct((_B, _D), jnp.float32),
            jax.ShapeDtypeStruct((_NW, _N_E), jnp.float32),
        ],
        scratch_types=[
            pltpu.VMEM((_BPW,), jnp.int32),
            pltpu.VMEM((_BPW, _D), jnp.float32),
            pltpu.VMEM((_N_E,), jnp.float32),
            pltpu.SemaphoreType.DMA,
        ],
        compiler_params=pltpu.CompilerParams(use_tc_tiling_on_sc=False,
                                             needs_layout_passes=False),
    )
    return f(emb, idx)


def _fin_body(z_ref, zq_ref, ap_ref, ent_ref, histsc_ref,
              out_ref, loss_ref, perp_ref, entmin_ref):
    z = z_ref[...]
    zn = z / jnp.maximum(jnp.sqrt(jnp.sum(z * z, axis=1, keepdims=True)), 1e-12)
    zq = zq_ref[...]
    zqn = zq / jnp.maximum(jnp.sqrt(jnp.sum(zq * zq, axis=1, keepdims=True)),
                           1e-12)
    out_ref[...] = zn + (zqn - zn)
    diff = zqn - zn
    commit = (1.0 + _BETA) * jnp.mean(diff * diff)
    ap = ap_ref[...] * (1.0 / _B)
    ent_max = -jnp.sum(ap * jnp.log(ap))
    hist = jnp.sum(histsc_ref[...], axis=0, keepdims=True)
    loss_ref[...] = (commit - ent_max).reshape(1, 1)
    probs = hist * (1.0 / _B)
    perp_ref[...] = jnp.exp(-jnp.sum(probs * jnp.log(probs + 1e-10))).reshape(1, 1)
    entmin_ref[...] = ent_ref[...] * (1.0 / _B)


def _finalize(z_flat, zq, ap_sum, ent_sum, hist_sc):
    return pl.pallas_call(
        _fin_body,
        out_shape=[
            jax.ShapeDtypeStruct((_B, _D), jnp.float32),
            jax.ShapeDtypeStruct((1, 1), jnp.float32),
            jax.ShapeDtypeStruct((1, 1), jnp.float32),
            jax.ShapeDtypeStruct((1, 1), jnp.float32),
        ],
    )(z_flat, zq, ap_sum, ent_sum, hist_sc)


def kernel(z, emb):
    z_flat = z.reshape(_B, _D)
    idx3, ap_sum, ent_sum = _main_tc(z_flat, emb.T)
    idx = idx3.reshape(_B)
    zq, hist_sc = _sc_gather(emb, idx)
    zq_out, loss, perp, entmin = _finalize(z_flat, zq, ap_sum, ent_sum,
                                           hist_sc)
    return (zq_out.reshape(z.shape), idx, loss[0, 0], perp[0, 0],
            entmin[0, 0])
